# baseline (speedup 1.0000x reference)
import functools

import jax
import jax.numpy as jnp
from jax import lax
from jax.experimental import pallas as pl
from jax.experimental.pallas import tpu as pltpu
from jax.experimental.pallas import tpu_sc as plsc

N_NODES = 10000
N_EDGES = 320000
CH = 128

NC = 2
NS = 16
NW = NC * NS

K = 128
CT = 2560
EPAD = CT * K
CHUNKS = CT // NW
PH = 40
NPAD = 10240
RPT = NPAD // NS
RBM = 2000

_mesh = plsc.VectorSubcoreMesh(core_axis_name="c", subcore_axis_name="s")


@functools.partial(
    pl.kernel,
    mesh=_mesh,
    out_type=jax.ShapeDtypeStruct((NC * NPAD,), jnp.float32),
    scratch_types=[
        pltpu.VMEM((CHUNKS, K), jnp.int32),
        pltpu.VMEM((K,), jnp.float32),
        pltpu.VMEM((RPT,), jnp.float32),
        pltpu.VMEM_SHARED((NPAD,), jnp.float32),
    ],
)
def _sc_degree(e2d_hbm, deg_hbm, idx_v, ones_v, zero_v, deg_sh):
    c = lax.axis_index("c")
    s = lax.axis_index("s")
    t = c * NS + s
    for i in range(K // 16):
        ones_v[pl.ds(i * 16, 16)] = jnp.ones((16,), jnp.float32)

    def zbody(i, carry):
        zero_v[pl.ds(i * 16, 16)] = jnp.zeros((16,), jnp.float32)
        return carry

    lax.fori_loop(0, RPT // 16, zbody, 0)
    pltpu.sync_copy(zero_v, deg_sh.at[pl.ds(s * RPT, RPT)])
    pltpu.sync_copy(e2d_hbm.at[pl.ds(CT + t * CHUNKS, CHUNKS)], idx_v)
    plsc.subcore_barrier()

    def body(j, carry):
        pltpu.sync_copy(ones_v, deg_sh.at[idx_v.at[j]], add=True)
        return carry

    lax.fori_loop(0, CHUNKS, body, 0)
    plsc.subcore_barrier()
    pltpu.sync_copy(deg_sh.at[pl.ds(s * RPT, RPT)],
                    deg_hbm.at[pl.ds(c * NPAD + s * RPT, RPT)])


@functools.partial(
    pl.kernel,
    mesh=_mesh,
    out_type=(jax.ShapeDtypeStruct((NPAD, CH), jnp.float32),
              jax.ShapeDtypeStruct((NPAD, CH), jnp.float32)),
    scratch_types=[
        pltpu.VMEM((PH, K), jnp.int32),
        pltpu.VMEM((PH, K), jnp.int32),
        pltpu.VMEM((K, CH), jnp.float32),
        pltpu.VMEM((K, CH), jnp.float32),
        pltpu.VMEM_SHARED((NPAD, CH), jnp.float32),
        pltpu.SemaphoreType.DMA,
        pltpu.SemaphoreType.DMA,
    ],
)
def _sc_edge_acc(y_hbm, e2d_hbm, acc0_hbm, acc1_hbm,
                 sidx, didx, rows0, rows1, acc_sh, sem0, sem1):
    c = lax.axis_index("c")
    s = lax.axis_index("s")
    t = c * NS + s

    def zbody(i, carry):
        for k in range(CH // 16):
            rows0[i, pl.ds(k * 16, 16)] = jnp.zeros((16,), jnp.float32)
        return carry

    lax.fori_loop(0, K, zbody, 0)
    for r in range(RPT // K):
        pltpu.sync_copy(rows0, acc_sh.at[pl.ds(s * RPT + r * K, K)])
    plsc.subcore_barrier()

    groups = PH // 2
    for ph in range(CHUNKS // PH):
        base = t * CHUNKS + ph * PH
        pltpu.sync_copy(e2d_hbm.at[pl.ds(base, PH)], sidx)
        pltpu.sync_copy(e2d_hbm.at[pl.ds(CT + base, PH)], didx)
        pltpu.make_async_copy(y_hbm.at[sidx.at[0]], rows0, sem0).start()

        def body(g, carry):
            j0 = 2 * g
            pltpu.make_async_copy(y_hbm.at[sidx.at[j0 + 1]], rows1, sem1).start()
            pltpu.make_async_copy(y_hbm.at[sidx.at[j0]], rows0, sem0).wait()
            pltpu.sync_copy(rows0, acc_sh.at[didx.at[j0]], add=True)

            @pl.when(g + 1 < groups)
            def _():
                pltpu.make_async_copy(y_hbm.at[sidx.at[j0 + 2]], rows0, sem0).start()

            pltpu.make_async_copy(y_hbm.at[sidx.at[j0 + 1]], rows1, sem1).wait()
            pltpu.sync_copy(rows1, acc_sh.at[didx.at[j0 + 1]], add=True)
            return carry

        lax.fori_loop(0, groups, body, 0)
    plsc.subcore_barrier()

    @pl.when(c == 0)
    def _():
        pltpu.sync_copy(acc_sh.at[pl.ds(s * RPT, RPT)],
                        acc0_hbm.at[pl.ds(s * RPT, RPT)])

    @pl.when(c == 1)
    def _():
        pltpu.sync_copy(acc_sh.at[pl.ds(s * RPT, RPT)],
                        acc1_hbm.at[pl.ds(s * RPT, RPT)])


def _mm_body(x_ref, w_ref, d0_ref, d1_ref, y_ref):
    dis = lax.rsqrt(d0_ref[...] + d1_ref[...] + 1.0)
    xw = jnp.dot(x_ref[...], w_ref[...], preferred_element_type=jnp.float32)
    y_ref[...] = xw * dis


def _fin_body(a0_ref, a1_ref, y_ref, d0_ref, d1_ref, b_ref, o_ref):
    dis = lax.rsqrt(d0_ref[...] + d1_ref[...] + 1.0)
    temp = dis * (a0_ref[...] + a1_ref[...] + y_ref[...]) + b_ref[...]
    o_ref[...] = temp + jnp.where(temp >= 0, temp, 0.01 * temp)


def kernel(x, edges, W, b):
    e32 = edges.astype(jnp.int32)
    padc = jnp.concatenate(
        [jnp.zeros((1, EPAD - N_EDGES), jnp.int32),
         jnp.full((1, EPAD - N_EDGES), N_NODES, jnp.int32)], axis=0)
    e2d = jnp.concatenate([e32, padc], axis=1).reshape(2 * CT, K)

    degp = _sc_degree(e2d)
    d0 = degp[:NPAD].reshape(NPAD, 1)
    d1 = degp[NPAD:].reshape(NPAD, 1)

    y = pl.pallas_call(
        _mm_body,
        grid=(N_NODES // RBM,),
        in_specs=[
            pl.BlockSpec((RBM, CH), lambda i: (i, 0)),
            pl.BlockSpec((CH, CH), lambda i: (0, 0)),
            pl.BlockSpec((RBM, 1), lambda i: (i, 0)),
            pl.BlockSpec((RBM, 1), lambda i: (i, 0)),
        ],
        out_specs=pl.BlockSpec((RBM, CH), lambda i: (i, 0)),
        out_shape=jax.ShapeDtypeStruct((N_NODES, CH), jnp.float32),
    )(x, W, d0, d1)

    acc0, acc1 = _sc_edge_acc(y, e2d)

    out = pl.pallas_call(
        _fin_body,
        grid=(N_NODES // RBM,),
        in_specs=[
            pl.BlockSpec((RBM, CH), lambda i: (i, 0)),
            pl.BlockSpec((RBM, CH), lambda i: (i, 0)),
            pl.BlockSpec((RBM, CH), lambda i: (i, 0)),
            pl.BlockSpec((RBM, 1), lambda i: (i, 0)),
            pl.BlockSpec((RBM, 1), lambda i: (i, 0)),
            pl.BlockSpec((1, CH), lambda i: (0, 0)),
        ],
        out_specs=pl.BlockSpec((RBM, CH), lambda i: (i, 0)),
        out_shape=jax.ShapeDtypeStruct((N_NODES, CH), jnp.float32),
    )(acc0, acc1, y, d0, d1, b.reshape(1, CH))

    return out


# speedup vs baseline: 1.1470x; 1.1470x over previous
import functools

import jax
import jax.numpy as jnp
from jax import lax
from jax.experimental import pallas as pl
from jax.experimental.pallas import tpu as pltpu
from jax.experimental.pallas import tpu_sc as plsc

N_NODES = 10000
N_EDGES = 320000
CH = 128

NC = 2
NS = 16
NW = NC * NS

K = 128
CT = 2560
EPAD = CT * K
CHUNKS = CT // NW
PH = 40
NPAD = 10240
RPT = NPAD // NS
RBM = 2000

_mesh = plsc.VectorSubcoreMesh(core_axis_name="c", subcore_axis_name="s")


@functools.partial(
    pl.kernel,
    mesh=_mesh,
    out_type=jax.ShapeDtypeStruct((NC * NPAD,), jnp.float32),
    scratch_types=[
        pltpu.VMEM((CHUNKS, K), jnp.int32),
        pltpu.VMEM((K,), jnp.float32),
        pltpu.VMEM((RPT,), jnp.float32),
        pltpu.VMEM_SHARED((NPAD,), jnp.float32),
    ],
)
def _sc_degree(e2d_hbm, deg_hbm, idx_v, ones_v, zero_v, deg_sh):
    c = lax.axis_index("c")
    s = lax.axis_index("s")
    t = c * NS + s
    for i in range(K // 16):
        ones_v[pl.ds(i * 16, 16)] = jnp.ones((16,), jnp.float32)

    def zbody(i, carry):
        zero_v[pl.ds(i * 16, 16)] = jnp.zeros((16,), jnp.float32)
        return carry

    lax.fori_loop(0, RPT // 16, zbody, 0)
    pltpu.sync_copy(zero_v, deg_sh.at[pl.ds(s * RPT, RPT)])
    pltpu.sync_copy(e2d_hbm.at[pl.ds(CT + t * CHUNKS, CHUNKS)], idx_v)
    plsc.subcore_barrier()

    def body(j, carry):
        pltpu.sync_copy(ones_v, deg_sh.at[idx_v.at[j]], add=True)
        return carry

    lax.fori_loop(0, CHUNKS, body, 0)
    plsc.subcore_barrier()
    pltpu.sync_copy(deg_sh.at[pl.ds(s * RPT, RPT)],
                    deg_hbm.at[pl.ds(c * NPAD + s * RPT, RPT)])


@functools.partial(
    pl.kernel,
    mesh=_mesh,
    out_type=jax.ShapeDtypeStruct((NC * NPAD, CH), jnp.float32),
    scratch_types=[
        pltpu.VMEM((PH, K), jnp.int32),
        pltpu.VMEM((PH, K), jnp.int32),
        pltpu.VMEM((K, CH), jnp.float32),
        pltpu.VMEM((K, CH), jnp.float32),
        pltpu.VMEM_SHARED((NPAD, CH), jnp.float32),
        pltpu.SemaphoreType.DMA,
        pltpu.SemaphoreType.DMA,
    ],
)
def _sc_edge_acc(y_hbm, e2d_hbm, acc_hbm,
                 sidx, didx, rows0, rows1, acc_sh, sem0, sem1):
    c = lax.axis_index("c")
    s = lax.axis_index("s")
    t = c * NS + s

    def zbody(i, carry):
        for k in range(CH // 16):
            rows0[i, pl.ds(k * 16, 16)] = jnp.zeros((16,), jnp.float32)
        return carry

    lax.fori_loop(0, K, zbody, 0)
    for r in range(RPT // K):
        pltpu.sync_copy(rows0, acc_sh.at[pl.ds(s * RPT + r * K, K)])
    plsc.subcore_barrier()

    groups = PH // 2
    for ph in range(CHUNKS // PH):
        base = t * CHUNKS + ph * PH
        pltpu.sync_copy(e2d_hbm.at[pl.ds(base, PH)], sidx)
        pltpu.sync_copy(e2d_hbm.at[pl.ds(CT + base, PH)], didx)
        pltpu.make_async_copy(y_hbm.at[sidx.at[0]], rows0, sem0).start()

        def body(g, carry):
            j0 = 2 * g
            pltpu.make_async_copy(y_hbm.at[sidx.at[j0 + 1]], rows1, sem1).start()
            pltpu.make_async_copy(y_hbm.at[sidx.at[j0]], rows0, sem0).wait()
            pltpu.sync_copy(rows0, acc_sh.at[didx.at[j0]], add=True)

            @pl.when(g + 1 < groups)
            def _():
                pltpu.make_async_copy(y_hbm.at[sidx.at[j0 + 2]], rows0, sem0).start()

            pltpu.make_async_copy(y_hbm.at[sidx.at[j0 + 1]], rows1, sem1).wait()
            pltpu.sync_copy(rows1, acc_sh.at[didx.at[j0 + 1]], add=True)
            return carry

        lax.fori_loop(0, groups, body, 0)
    plsc.subcore_barrier()
    pltpu.sync_copy(acc_sh.at[pl.ds(s * RPT, RPT)],
                    acc_hbm.at[pl.ds(c * NPAD + s * RPT, RPT)])


def _mm_body(x_ref, w_ref, d0_ref, d1_ref, y_ref):
    dis = lax.rsqrt(d0_ref[...] + d1_ref[...] + 1.0)
    xw = jnp.dot(x_ref[...], w_ref[...], preferred_element_type=jnp.float32)
    y_ref[...] = xw * dis


def _fin_body(a0_ref, a1_ref, y_ref, d0_ref, d1_ref, b_ref, o_ref):
    dis = lax.rsqrt(d0_ref[...] + d1_ref[...] + 1.0)
    temp = dis * (a0_ref[...] + a1_ref[...] + y_ref[...]) + b_ref[...]
    o_ref[...] = temp + jnp.where(temp >= 0, temp, 0.01 * temp)


def kernel(x, edges, W, b):
    e32 = edges.astype(jnp.int32)
    padc = jnp.concatenate(
        [jnp.zeros((1, EPAD - N_EDGES), jnp.int32),
         jnp.full((1, EPAD - N_EDGES), N_NODES, jnp.int32)], axis=0)
    e2d = jnp.concatenate([e32, padc], axis=1).reshape(2 * CT, K)

    degp = _sc_degree(e2d)
    d0 = degp[:NPAD].reshape(NPAD, 1)
    d1 = degp[NPAD:].reshape(NPAD, 1)

    y = pl.pallas_call(
        _mm_body,
        grid=(N_NODES // RBM,),
        in_specs=[
            pl.BlockSpec((RBM, CH), lambda i: (i, 0)),
            pl.BlockSpec((CH, CH), lambda i: (0, 0)),
            pl.BlockSpec((RBM, 1), lambda i: (i, 0)),
            pl.BlockSpec((RBM, 1), lambda i: (i, 0)),
        ],
        out_specs=pl.BlockSpec((RBM, CH), lambda i: (i, 0)),
        out_shape=jax.ShapeDtypeStruct((N_NODES, CH), jnp.float32),
    )(x, W, d0, d1)

    accp = _sc_edge_acc(y, e2d)
    acc0 = accp[:NPAD]
    acc1 = accp[NPAD:]

    out = pl.pallas_call(
        _fin_body,
        grid=(N_NODES // RBM,),
        in_specs=[
            pl.BlockSpec((RBM, CH), lambda i: (i, 0)),
            pl.BlockSpec((RBM, CH), lambda i: (i, 0)),
            pl.BlockSpec((RBM, CH), lambda i: (i, 0)),
            pl.BlockSpec((RBM, 1), lambda i: (i, 0)),
            pl.BlockSpec((RBM, 1), lambda i: (i, 0)),
            pl.BlockSpec((1, CH), lambda i: (0, 0)),
        ],
        out_specs=pl.BlockSpec((RBM, CH), lambda i: (i, 0)),
        out_shape=jax.ShapeDtypeStruct((N_NODES, CH), jnp.float32),
    )(acc0, acc1, y, d0, d1, b.reshape(1, CH))

    return out


# speedup vs baseline: 3.0085x; 2.6230x over previous
import functools

import jax
import jax.numpy as jnp
from jax import lax
from jax.experimental import pallas as pl
from jax.experimental.pallas import tpu as pltpu
from jax.experimental.pallas import tpu_sc as plsc

N_NODES = 10000
N_EDGES = 320000
CH = 128

NC = 2
NS = 16
NW = NC * NS

K = 125
CT = 2560
CHUNKS = CT // NW
PH = 40
NPAD = 10240
RPT = NPAD // NS
RBM = 2000

_mesh = plsc.VectorSubcoreMesh(core_axis_name="c", subcore_axis_name="s")


@functools.partial(
    pl.kernel,
    mesh=_mesh,
    out_type=jax.ShapeDtypeStruct((NC * NPAD,), jnp.float32),
    scratch_types=[
        pltpu.VMEM((CHUNKS, K), jnp.int32),
        pltpu.VMEM((128,), jnp.float32),
        pltpu.VMEM((RPT,), jnp.float32),
        pltpu.VMEM_SHARED((NPAD,), jnp.float32),
    ],
)
def _sc_degree(e2d_hbm, deg_hbm, idx_v, ones_v, zero_v, deg_sh):
    c = lax.axis_index("c")
    s = lax.axis_index("s")
    t = c * NS + s
    for i in range(128 // 16):
        ones_v[pl.ds(i * 16, 16)] = jnp.ones((16,), jnp.float32)

    def zbody(i, carry):
        zero_v[pl.ds(i * 16, 16)] = jnp.zeros((16,), jnp.float32)
        return carry

    lax.fori_loop(0, RPT // 16, zbody, 0)
    pltpu.sync_copy(zero_v, deg_sh.at[pl.ds(s * RPT, RPT)])
    pltpu.sync_copy(e2d_hbm.at[pl.ds(CT + t * CHUNKS, CHUNKS)], idx_v)
    plsc.subcore_barrier()

    def body(j, carry):
        pltpu.sync_copy(ones_v.at[pl.ds(0, K)], deg_sh.at[idx_v.at[j]], add=True)
        return carry

    lax.fori_loop(0, CHUNKS, body, 0)
    plsc.subcore_barrier()
    pltpu.sync_copy(deg_sh.at[pl.ds(s * RPT, RPT)],
                    deg_hbm.at[pl.ds(c * NPAD + s * RPT, RPT)])


@functools.partial(
    pl.kernel,
    mesh=_mesh,
    out_type=jax.ShapeDtypeStruct((NC * NPAD, CH), jnp.float32),
    scratch_types=[
        pltpu.VMEM((PH, K), jnp.int32),
        pltpu.VMEM((PH, K), jnp.int32),
        pltpu.VMEM((128, CH), jnp.float32),
        pltpu.VMEM((128, CH), jnp.float32),
        pltpu.VMEM_SHARED((NPAD, CH), jnp.float32),
        pltpu.SemaphoreType.DMA,
        pltpu.SemaphoreType.DMA,
    ],
)
def _sc_edge_acc(y_hbm, e2d_hbm, acc_hbm,
                 sidx, didx, rows0, rows1, acc_sh, sem0, sem1):
    c = lax.axis_index("c")
    s = lax.axis_index("s")
    t = c * NS + s

    def zbody(i, carry):
        for k in range(CH // 16):
            rows0[i, pl.ds(k * 16, 16)] = jnp.zeros((16,), jnp.float32)
        return carry

    lax.fori_loop(0, 128, zbody, 0)
    for r in range(RPT // 128):
        pltpu.sync_copy(rows0, acc_sh.at[pl.ds(s * RPT + r * 128, 128)])
    plsc.subcore_barrier()

    b0 = rows0.at[pl.ds(0, K)]
    b1 = rows1.at[pl.ds(0, K)]
    groups = PH // 2
    for ph in range(CHUNKS // PH):
        base = t * CHUNKS + ph * PH
        pltpu.sync_copy(e2d_hbm.at[pl.ds(base, PH)], sidx)
        pltpu.sync_copy(e2d_hbm.at[pl.ds(CT + base, PH)], didx)
        pltpu.make_async_copy(y_hbm.at[sidx.at[0]], b0, sem0).start()

        def body(g, carry):
            j0 = 2 * g
            pltpu.make_async_copy(y_hbm.at[sidx.at[j0 + 1]], b1, sem1).start()
            pltpu.make_async_copy(y_hbm.at[sidx.at[j0]], b0, sem0).wait()
            pltpu.sync_copy(b0, acc_sh.at[didx.at[j0]], add=True)

            @pl.when(g + 1 < groups)
            def _():
                pltpu.make_async_copy(y_hbm.at[sidx.at[j0 + 2]], b0, sem0).start()

            pltpu.make_async_copy(y_hbm.at[sidx.at[j0 + 1]], b1, sem1).wait()
            pltpu.sync_copy(b1, acc_sh.at[didx.at[j0 + 1]], add=True)
            return carry

        lax.fori_loop(0, groups, body, 0)
    plsc.subcore_barrier()
    pltpu.sync_copy(acc_sh.at[pl.ds(s * RPT, RPT)],
                    acc_hbm.at[pl.ds(c * NPAD + s * RPT, RPT)])


def _mm_body(x_ref, w_ref, d0_ref, d1_ref, y_ref):
    dis = lax.rsqrt(d0_ref[...] + d1_ref[...] + 1.0)
    xw = jnp.dot(x_ref[...], w_ref[...], preferred_element_type=jnp.float32)
    y_ref[...] = xw * dis


def _fin_body(a0_ref, a1_ref, y_ref, d0_ref, d1_ref, b_ref, o_ref):
    dis = lax.rsqrt(d0_ref[...] + d1_ref[...] + 1.0)
    temp = dis * (a0_ref[...] + a1_ref[...] + y_ref[...]) + b_ref[...]
    o_ref[...] = temp + jnp.where(temp >= 0, temp, 0.01 * temp)


def kernel(x, edges, W, b):
    e2d = edges.astype(jnp.int32).reshape(2 * CT, K)

    degp = _sc_degree(e2d)
    d0 = degp[:NPAD].reshape(NPAD, 1)
    d1 = degp[NPAD:].reshape(NPAD, 1)

    y = pl.pallas_call(
        _mm_body,
        grid=(N_NODES // RBM,),
        in_specs=[
            pl.BlockSpec((RBM, CH), lambda i: (i, 0)),
            pl.BlockSpec((CH, CH), lambda i: (0, 0)),
            pl.BlockSpec((RBM, 1), lambda i: (i, 0)),
            pl.BlockSpec((RBM, 1), lambda i: (i, 0)),
        ],
        out_specs=pl.BlockSpec((RBM, CH), lambda i: (i, 0)),
        out_shape=jax.ShapeDtypeStruct((N_NODES, CH), jnp.float32),
    )(x, W, d0, d1)

    accp = _sc_edge_acc(y, e2d)

    out = pl.pallas_call(
        _fin_body,
        grid=(N_NODES // RBM,),
        in_specs=[
            pl.BlockSpec((RBM, CH), lambda i: (i, 0)),
            pl.BlockSpec((RBM, CH), lambda i: (i, 0)),
            pl.BlockSpec((RBM, CH), lambda i: (i, 0)),
            pl.BlockSpec((RBM, 1), lambda i: (i, 0)),
            pl.BlockSpec((RBM, 1), lambda i: (i, 0)),
            pl.BlockSpec((1, CH), lambda i: (0, 0)),
        ],
        out_specs=pl.BlockSpec((RBM, CH), lambda i: (i, 0)),
        out_shape=jax.ShapeDtypeStruct((N_NODES, CH), jnp.float32),
    )(accp[:NPAD], accp[NPAD:], y, d0, d1, b.reshape(1, CH))

    return out


# speedup vs baseline: 3.1485x; 1.0465x over previous
import functools

import jax
import jax.numpy as jnp
from jax import lax
from jax.experimental import pallas as pl
from jax.experimental.pallas import tpu as pltpu
from jax.experimental.pallas import tpu_sc as plsc

N_NODES = 10000
N_EDGES = 320000
CH = 128

NC = 2
NS = 16
NW = NC * NS

K = 125
CT = 2560
CHUNKS = CT // NW
PH = 40
NPAD = 10240
RPT = NPAD // NS
RBM = 2000

_mesh = plsc.VectorSubcoreMesh(core_axis_name="c", subcore_axis_name="s")


@functools.partial(
    pl.kernel,
    mesh=_mesh,
    out_type=jax.ShapeDtypeStruct((NC * NPAD,), jnp.float32),
    scratch_types=[
        pltpu.VMEM((CHUNKS, K), jnp.int32),
        pltpu.VMEM((128,), jnp.float32),
        pltpu.VMEM((RPT,), jnp.float32),
        pltpu.VMEM_SHARED((NPAD,), jnp.float32),
    ],
)
def _sc_degree(e2d_hbm, deg_hbm, idx_v, ones_v, zero_v, deg_sh):
    c = lax.axis_index("c")
    s = lax.axis_index("s")
    t = c * NS + s
    for i in range(128 // 16):
        ones_v[pl.ds(i * 16, 16)] = jnp.ones((16,), jnp.float32)

    def zbody(i, carry):
        zero_v[pl.ds(i * 16, 16)] = jnp.zeros((16,), jnp.float32)
        return carry

    lax.fori_loop(0, RPT // 16, zbody, 0)
    pltpu.sync_copy(zero_v, deg_sh.at[pl.ds(s * RPT, RPT)])
    pltpu.sync_copy(e2d_hbm.at[pl.ds(CT + t * CHUNKS, CHUNKS)], idx_v)
    plsc.subcore_barrier()

    def body(j, carry):
        pltpu.sync_copy(ones_v.at[pl.ds(0, K)], deg_sh.at[idx_v.at[j]], add=True)
        return carry

    lax.fori_loop(0, CHUNKS, body, 0)
    plsc.subcore_barrier()
    pltpu.sync_copy(deg_sh.at[pl.ds(s * RPT, RPT)],
                    deg_hbm.at[pl.ds(c * NPAD + s * RPT, RPT)])


@functools.partial(
    pl.kernel,
    mesh=_mesh,
    out_type=(jax.ShapeDtypeStruct((NPAD, CH), jnp.float32),
              jax.ShapeDtypeStruct((NPAD, CH), jnp.float32)),
    scratch_types=[
        pltpu.VMEM((PH, K), jnp.int32),
        pltpu.VMEM((PH, K), jnp.int32),
        pltpu.VMEM((128, CH), jnp.float32),
        pltpu.VMEM((128, CH), jnp.float32),
        pltpu.VMEM_SHARED((NPAD, CH), jnp.float32),
        pltpu.SemaphoreType.DMA,
        pltpu.SemaphoreType.DMA,
    ],
)
def _sc_edge_acc(y_hbm, e2d_hbm, acc0_hbm, acc1_hbm,
                 sidx, didx, rows0, rows1, acc_sh, sem0, sem1):
    c = lax.axis_index("c")
    s = lax.axis_index("s")
    t = c * NS + s

    def zbody(i, carry):
        for k in range(CH // 16):
            rows0[i, pl.ds(k * 16, 16)] = jnp.zeros((16,), jnp.float32)
        return carry

    lax.fori_loop(0, 128, zbody, 0)
    for r in range(RPT // 128):
        pltpu.sync_copy(rows0, acc_sh.at[pl.ds(s * RPT + r * 128, 128)])
    plsc.subcore_barrier()

    b0 = rows0.at[pl.ds(0, K)]
    b1 = rows1.at[pl.ds(0, K)]
    groups = PH // 2
    for ph in range(CHUNKS // PH):
        base = t * CHUNKS + ph * PH
        pltpu.sync_copy(e2d_hbm.at[pl.ds(base, PH)], sidx)
        pltpu.sync_copy(e2d_hbm.at[pl.ds(CT + base, PH)], didx)
        pltpu.make_async_copy(y_hbm.at[sidx.at[0]], b0, sem0).start()

        def body(g, carry):
            j0 = 2 * g
            pltpu.make_async_copy(y_hbm.at[sidx.at[j0 + 1]], b1, sem1).start()
            pltpu.make_async_copy(y_hbm.at[sidx.at[j0]], b0, sem0).wait()
            pltpu.sync_copy(b0, acc_sh.at[didx.at[j0]], add=True)

            @pl.when(g + 1 < groups)
            def _():
                pltpu.make_async_copy(y_hbm.at[sidx.at[j0 + 2]], b0, sem0).start()

            pltpu.make_async_copy(y_hbm.at[sidx.at[j0 + 1]], b1, sem1).wait()
            pltpu.sync_copy(b1, acc_sh.at[didx.at[j0 + 1]], add=True)
            return carry

        lax.fori_loop(0, groups, body, 0)
    plsc.subcore_barrier()

    @pl.when(c == 0)
    def _():
        pltpu.sync_copy(acc_sh.at[pl.ds(s * RPT, RPT)],
                        acc0_hbm.at[pl.ds(s * RPT, RPT)])

    @pl.when(c == 1)
    def _():
        pltpu.sync_copy(acc_sh.at[pl.ds(s * RPT, RPT)],
                        acc1_hbm.at[pl.ds(s * RPT, RPT)])


def _mm_body(x_ref, w_ref, d0_ref, d1_ref, y_ref):
    dis = lax.rsqrt(d0_ref[...] + d1_ref[...] + 1.0)
    xw = jnp.dot(x_ref[...], w_ref[...], preferred_element_type=jnp.float32)
    y_ref[...] = xw * dis


def _fin_body(a0_ref, a1_ref, y_ref, d0_ref, d1_ref, b_ref, o_ref):
    dis = lax.rsqrt(d0_ref[...] + d1_ref[...] + 1.0)
    temp = dis * (a0_ref[...] + a1_ref[...] + y_ref[...]) + b_ref[...]
    o_ref[...] = temp + jnp.where(temp >= 0, temp, 0.01 * temp)


def kernel(x, edges, W, b):
    e2d = edges.astype(jnp.int32).reshape(2 * CT, K)

    degp = _sc_degree(e2d)
    d0 = degp[:NPAD].reshape(NPAD, 1)
    d1 = degp[NPAD:].reshape(NPAD, 1)

    y = pl.pallas_call(
        _mm_body,
        grid=(N_NODES // RBM,),
        in_specs=[
            pl.BlockSpec((RBM, CH), lambda i: (i, 0)),
            pl.BlockSpec((CH, CH), lambda i: (0, 0)),
            pl.BlockSpec((RBM, 1), lambda i: (i, 0)),
            pl.BlockSpec((RBM, 1), lambda i: (i, 0)),
        ],
        out_specs=pl.BlockSpec((RBM, CH), lambda i: (i, 0)),
        out_shape=jax.ShapeDtypeStruct((N_NODES, CH), jnp.float32),
    )(x, W, d0, d1)

    acc0, acc1 = _sc_edge_acc(y, e2d)

    out = pl.pallas_call(
        _fin_body,
        grid=(N_NODES // RBM,),
        in_specs=[
            pl.BlockSpec((RBM, CH), lambda i: (i, 0)),
            pl.BlockSpec((RBM, CH), lambda i: (i, 0)),
            pl.BlockSpec((RBM, CH), lambda i: (i, 0)),
            pl.BlockSpec((RBM, 1), lambda i: (i, 0)),
            pl.BlockSpec((RBM, 1), lambda i: (i, 0)),
            pl.BlockSpec((1, CH), lambda i: (0, 0)),
        ],
        out_specs=pl.BlockSpec((RBM, CH), lambda i: (i, 0)),
        out_shape=jax.ShapeDtypeStruct((N_NODES, CH), jnp.float32),
    )(acc0, acc1, y, d0, d1, b.reshape(1, CH))

    return out
